# baseline (speedup 1.0000x reference)
import jax
import jax.numpy as jnp
from jax.experimental import pallas as pl
from jax.experimental.pallas import tpu as pltpu

DEPTH = 1000
R = 1024
NBUF = 8
M = 20
N = 4096
NI = N // R
CHUNKS = M * NI


def _body(idx_ref, out_ref, buf_ref, sems):
    def make_copy(chunk, slot):
        j = chunk // NI
        i = chunk % NI
        return pltpu.make_async_copy(
            buf_ref.at[slot],
            out_ref.at[pl.ds(j, 1), :, pl.ds(i * R, R)],
            sems.at[slot],
        )

    iota = jax.lax.broadcasted_iota(jnp.int32, (1, DEPTH, R), 1)

    def group(g, carry):
        for slot in range(NBUF):
            chunk = g * NBUF + slot
            j = chunk // NI
            i = chunk % NI

            @pl.when(g > 0)
            def _wait_prev():
                make_copy(chunk - NBUF, slot).wait()

            idx = idx_ref[pl.ds(j, 1), :, pl.ds(i * R, R)]
            buf_ref[slot] = (idx == iota).astype(jnp.float32)
            make_copy(chunk, slot).start()
        return carry

    jax.lax.fori_loop(0, CHUNKS // NBUF, group, 0)
    for slot in range(NBUF):
        make_copy(CHUNKS - NBUF + slot, slot).wait()


def kernel(inputs):
    n, m = inputs.shape
    idx_t = inputs.T.reshape(m, 1, n)
    out_t = pl.pallas_call(
        _body,
        in_specs=[pl.BlockSpec(memory_space=pltpu.MemorySpace.VMEM)],
        out_specs=pl.BlockSpec(memory_space=pl.ANY),
        out_shape=jax.ShapeDtypeStruct((m, DEPTH, n), jnp.float32),
        scratch_shapes=[
            pltpu.VMEM((NBUF, 1, DEPTH, R), jnp.float32),
            pltpu.SemaphoreType.DMA((NBUF,)),
        ],
        compiler_params=pltpu.CompilerParams(
            vmem_limit_bytes=100 * 1024 * 1024,
        ),
    )(idx_t)
    return out_t.transpose(2, 0, 1)
